# baseline (speedup 1.0000x reference)
import functools

import jax
import jax.numpy as jnp
from jax import lax
from jax.experimental import pallas as pl
from jax.experimental.pallas import tpu as pltpu
from jax.experimental.pallas import tpu_sc as plsc

MAXW = 1000000
D = 32
H = 1000
C = 1000
B = 16384
L = 50
EPS = 1e-5

NC = 2
NS = 16
NW = NC * NS
BPW = B // NW
NT = BPW
NBUF = 4
LP = 64
VR = 128


def _pool_body(x_hbm, table_hbm, h_hbm, idx_v, rvr_v, rows_v, h_v, sems):
    wid = lax.axis_index("s") * NC + lax.axis_index("c")
    pltpu.sync_copy(x_hbm.at[pl.ds(wid * BPW * LP, BPW * LP)], idx_v)

    def fire(t, b):
        for k in range(LP // 16):
            rvr_v[b, pl.ds(k * 16, 16)] = lax.shift_right_logical(
                idx_v[pl.ds(t * LP + k * 16, 16)], 2)
        pltpu.async_copy(table_hbm.at[rvr_v.at[b]], rows_v.at[b], sems.at[b])

    for b in range(NBUF):
        fire(b, b)

    def outer(g, carry):
        for b in range(NBUF):
            t = g * NBUF + b
            pltpu.make_async_copy(
                table_hbm.at[rvr_v.at[0]], rows_v.at[b], sems.at[b]).wait()
            ov = [(idx_v[pl.ds(t * LP + k * 16, 16)] & 3) * 32
                  for k in range(4)]
            o = ov[0][0]
            acc0 = rows_v[b, 0, pl.ds(o, 16)]
            acc1 = rows_v[b, 0, pl.ds(o + 16, 16)]
            for l in range(1, L):
                o = ov[l // 16][l % 16]
                acc0 = acc0 + rows_v[b, l, pl.ds(o, 16)]
                acc1 = acc1 + rows_v[b, l, pl.ds(o + 16, 16)]
            h_v[pl.ds(t * D, 16)] = acc0
            h_v[pl.ds(t * D + 16, 16)] = acc1
            tn = t + NBUF

            @pl.when(tn < NT)
            def _():
                fire(tn, b)
        return carry

    lax.fori_loop(0, NT // NBUF, outer, 0)
    pltpu.sync_copy(h_v, h_hbm.at[pl.ds(wid * BPW * D, BPW * D)])


@functools.partial(jax.jit, static_argnames=())
def _pool(x1d, tabv):
    mesh = plsc.VectorSubcoreMesh(core_axis_name="c", subcore_axis_name="s")
    return pl.kernel(
        _pool_body,
        out_type=jax.ShapeDtypeStruct((B * D,), jnp.float32),
        mesh=mesh,
        scratch_types=[
            pltpu.VMEM((BPW * LP,), jnp.int32),
            pltpu.VMEM((NBUF, LP), jnp.int32),
            pltpu.VMEM((NBUF, LP, VR), jnp.float32),
            pltpu.VMEM((BPW * D,), jnp.float32),
            pltpu.SemaphoreType.DMA((NBUF,)),
        ],
        compiler_params=pltpu.CompilerParams(use_tc_tiling_on_sc=True),
    )(x1d, tabv)


BB1 = 2048
BBL = 1024


def _k1_body(h_ref, w_ref, b_ref, z_ref, s_ref):
    z = jnp.dot(h_ref[...], w_ref[...],
                preferred_element_type=jnp.float32) + b_ref[...]
    z_ref[...] = z.astype(jnp.bfloat16)
    acc = jnp.concatenate(
        [jnp.sum(z, axis=0, keepdims=True),
         jnp.sum(z * z, axis=0, keepdims=True)], axis=0)

    @pl.when(pl.program_id(0) == 0)
    def _():
        s_ref[...] = acc

    @pl.when(pl.program_id(0) != 0)
    def _():
        s_ref[...] += acc


def _layer_body(z_ref, s_ref, g_ref, be_ref, w_ref, b_ref, zn_ref, so_ref):
    s = s_ref[...]
    mu = s[0:1, :] * (1.0 / B)
    var = s[1:2, :] * (1.0 / B) - mu * mu
    inv = g_ref[...] * lax.rsqrt(var + EPS)
    a = jnp.maximum((z_ref[...].astype(jnp.float32) - mu) * inv
                    + be_ref[...], 0.0)
    zn = jnp.dot(a.astype(jnp.bfloat16), w_ref[...],
                 preferred_element_type=jnp.float32) + b_ref[...]
    zn_ref[...] = zn.astype(jnp.bfloat16)
    acc = jnp.concatenate(
        [jnp.sum(zn, axis=0, keepdims=True),
         jnp.sum(zn * zn, axis=0, keepdims=True)], axis=0)

    @pl.when(pl.program_id(0) == 0)
    def _():
        so_ref[...] = acc

    @pl.when(pl.program_id(0) != 0)
    def _():
        so_ref[...] += acc


def _final_body(z_ref, s_ref, g_ref, be_ref, w_ref, b_ref, o_ref):
    s = s_ref[...]
    mu = s[0:1, :] * (1.0 / B)
    var = s[1:2, :] * (1.0 / B) - mu * mu
    inv = g_ref[...] * lax.rsqrt(var + EPS)
    a = jnp.maximum((z_ref[...].astype(jnp.float32) - mu) * inv
                    + be_ref[...], 0.0)
    ot = lax.dot_general(w_ref[...], a.astype(jnp.bfloat16),
                         (((1,), (1,)), ((), ())),
                         preferred_element_type=jnp.float32)
    o_ref[...] = ot + b_ref[...]


def _row_spec(bb, cols):
    return pl.BlockSpec((bb, cols), lambda i: (i, 0))


def _full_spec(rows, cols):
    return pl.BlockSpec((rows, cols), lambda i: (0, 0))


def _k1(h, w1t, b1):
    return pl.pallas_call(
        _k1_body,
        grid=(B // BB1,),
        in_specs=[_row_spec(BB1, D), _full_spec(D, H), _full_spec(1, H)],
        out_specs=[_row_spec(BB1, H), _full_spec(2, H)],
        out_shape=[jax.ShapeDtypeStruct((B, H), jnp.bfloat16),
                   jax.ShapeDtypeStruct((2, H), jnp.float32)],
    )(h, w1t, b1)


def _klayer(z, s, g, be, wt, b):
    return pl.pallas_call(
        _layer_body,
        grid=(B // BBL,),
        in_specs=[_row_spec(BBL, H), _full_spec(2, H), _full_spec(1, H),
                  _full_spec(1, H), _full_spec(H, H), _full_spec(1, H)],
        out_specs=[_row_spec(BBL, H), _full_spec(2, H)],
        out_shape=[jax.ShapeDtypeStruct((B, H), jnp.bfloat16),
                   jax.ShapeDtypeStruct((2, H), jnp.float32)],
    )(z, s, g, be, wt, b)


def _kfinal(z, s, g, be, w4, b4c):
    return pl.pallas_call(
        _final_body,
        grid=(B // BBL,),
        in_specs=[_row_spec(BBL, H), _full_spec(2, H), _full_spec(1, H),
                  _full_spec(1, H), _full_spec(C, H), _full_spec(C, 1)],
        out_specs=pl.BlockSpec((C, BBL), lambda i: (0, i)),
        out_shape=jax.ShapeDtypeStruct((C, B), jnp.float32),
    )(z, s, g, be, w4, b4c)


def kernel(x, table, W1, b1, g1, be1, W2, b2, g2, be2, W3, b3, g3, be3,
           W4, b4):
    bf = jnp.bfloat16
    x64 = jnp.pad(x, ((0, 0), (0, LP - L))).reshape(B * LP)
    h = _pool(x64, table.reshape(MAXW // 4, VR)).reshape(B, D)
    z1, s1 = _k1(h, W1.T, b1.reshape(1, H))
    z2, s2 = _klayer(z1, s1, g1.reshape(1, H), be1.reshape(1, H),
                     W2.T.astype(bf), b2.reshape(1, H))
    z3, s3 = _klayer(z2, s2, g2.reshape(1, H), be2.reshape(1, H),
                     W3.T.astype(bf), b3.reshape(1, H))
    ot = _kfinal(z3, s3, g3.reshape(1, H), be3.reshape(1, H),
                 W4.astype(bf), b4.reshape(C, 1))
    return ot.T


# speedup vs baseline: 12.9938x; 12.9938x over previous
import functools

import jax
import jax.numpy as jnp
from jax import lax
from jax.experimental import pallas as pl
from jax.experimental.pallas import tpu as pltpu
from jax.experimental.pallas import tpu_sc as plsc

MAXW = 1000000
D = 32
H = 1000
C = 1000
B = 16384
L = 50
EPS = 1e-5

NC = 2
NS = 16
NW = NC * NS
BPW = B // NW
NT = BPW
NBUF = 8


def _pool_body(x_hbm, table_hbm, h_hbm, idx_v, rows_v, h_v, sems):
    wid = lax.axis_index("s") * NC + lax.axis_index("c")
    pltpu.sync_copy(x_hbm.at[pl.ds(wid * BPW, BPW)], idx_v)

    def fire(t, b):
        pltpu.async_copy(table_hbm.at[idx_v.at[t]], rows_v.at[b], sems.at[b])

    for b in range(NBUF):
        fire(b, b)

    def outer(g, carry):
        for b in range(NBUF):
            t = g * NBUF + b
            pltpu.make_async_copy(
                table_hbm.at[idx_v.at[0]], rows_v.at[b], sems.at[b]).wait()
            acc0 = rows_v[b, 0, pl.ds(0, 16)]
            acc1 = rows_v[b, 0, pl.ds(16, 16)]
            for l in range(1, L):
                acc0 = acc0 + rows_v[b, l, pl.ds(0, 16)]
                acc1 = acc1 + rows_v[b, l, pl.ds(16, 16)]
            h_v[t, pl.ds(0, 16)] = acc0
            h_v[t, pl.ds(16, 16)] = acc1
            tn = t + NBUF

            @pl.when(tn < NT)
            def _():
                fire(tn, b)
        return carry

    lax.fori_loop(0, NT // NBUF, outer, 0)
    pltpu.sync_copy(h_v, h_hbm.at[pl.ds(wid * BPW, BPW)])


@functools.partial(jax.jit, static_argnames=())
def _pool(x, table):
    mesh = plsc.VectorSubcoreMesh(core_axis_name="c", subcore_axis_name="s")
    return pl.kernel(
        _pool_body,
        out_type=jax.ShapeDtypeStruct((B, D), jnp.float32),
        mesh=mesh,
        scratch_types=[
            pltpu.VMEM((BPW, L), jnp.int32),
            pltpu.VMEM((NBUF, L, D), jnp.float32),
            pltpu.VMEM((BPW, D), jnp.float32),
            pltpu.SemaphoreType.DMA((NBUF,)),
        ],
        compiler_params=pltpu.CompilerParams(use_tc_tiling_on_sc=False),
    )(x, table)


BB1 = 2048
BBL = 1024


def _k1_body(h_ref, w_ref, b_ref, z_ref, s_ref):
    z = jnp.dot(h_ref[...], w_ref[...],
                preferred_element_type=jnp.float32) + b_ref[...]
    z_ref[...] = z.astype(jnp.bfloat16)
    acc = jnp.concatenate(
        [jnp.sum(z, axis=0, keepdims=True),
         jnp.sum(z * z, axis=0, keepdims=True)], axis=0)

    @pl.when(pl.program_id(0) == 0)
    def _():
        s_ref[...] = acc

    @pl.when(pl.program_id(0) != 0)
    def _():
        s_ref[...] += acc


def _layer_body(z_ref, s_ref, g_ref, be_ref, w_ref, b_ref, zn_ref, so_ref):
    s = s_ref[...]
    mu = s[0:1, :] * (1.0 / B)
    var = s[1:2, :] * (1.0 / B) - mu * mu
    inv = g_ref[...] * lax.rsqrt(var + EPS)
    a = jnp.maximum((z_ref[...].astype(jnp.float32) - mu) * inv
                    + be_ref[...], 0.0)
    zn = jnp.dot(a.astype(jnp.bfloat16), w_ref[...],
                 preferred_element_type=jnp.float32) + b_ref[...]
    zn_ref[...] = zn.astype(jnp.bfloat16)
    acc = jnp.concatenate(
        [jnp.sum(zn, axis=0, keepdims=True),
         jnp.sum(zn * zn, axis=0, keepdims=True)], axis=0)

    @pl.when(pl.program_id(0) == 0)
    def _():
        so_ref[...] = acc

    @pl.when(pl.program_id(0) != 0)
    def _():
        so_ref[...] += acc


def _final_body(z_ref, s_ref, g_ref, be_ref, w_ref, b_ref, o_ref):
    s = s_ref[...]
    mu = s[0:1, :] * (1.0 / B)
    var = s[1:2, :] * (1.0 / B) - mu * mu
    inv = g_ref[...] * lax.rsqrt(var + EPS)
    a = jnp.maximum((z_ref[...].astype(jnp.float32) - mu) * inv
                    + be_ref[...], 0.0)
    ot = lax.dot_general(w_ref[...], a.astype(jnp.bfloat16),
                         (((1,), (1,)), ((), ())),
                         preferred_element_type=jnp.float32)
    o_ref[...] = ot + b_ref[...]


def _row_spec(bb, cols):
    return pl.BlockSpec((bb, cols), lambda i: (i, 0))


def _full_spec(rows, cols):
    return pl.BlockSpec((rows, cols), lambda i: (0, 0))


def _k1(h, w1t, b1):
    return pl.pallas_call(
        _k1_body,
        grid=(B // BB1,),
        in_specs=[_row_spec(BB1, D), _full_spec(D, H), _full_spec(1, H)],
        out_specs=[_row_spec(BB1, H), _full_spec(2, H)],
        out_shape=[jax.ShapeDtypeStruct((B, H), jnp.bfloat16),
                   jax.ShapeDtypeStruct((2, H), jnp.float32)],
    )(h, w1t, b1)


def _klayer(z, s, g, be, wt, b):
    return pl.pallas_call(
        _layer_body,
        grid=(B // BBL,),
        in_specs=[_row_spec(BBL, H), _full_spec(2, H), _full_spec(1, H),
                  _full_spec(1, H), _full_spec(H, H), _full_spec(1, H)],
        out_specs=[_row_spec(BBL, H), _full_spec(2, H)],
        out_shape=[jax.ShapeDtypeStruct((B, H), jnp.bfloat16),
                   jax.ShapeDtypeStruct((2, H), jnp.float32)],
    )(z, s, g, be, wt, b)


def _kfinal(z, s, g, be, w4, b4c):
    return pl.pallas_call(
        _final_body,
        grid=(B // BBL,),
        in_specs=[_row_spec(BBL, H), _full_spec(2, H), _full_spec(1, H),
                  _full_spec(1, H), _full_spec(C, H), _full_spec(C, 1)],
        out_specs=pl.BlockSpec((C, BBL), lambda i: (0, i)),
        out_shape=jax.ShapeDtypeStruct((C, B), jnp.float32),
    )(z, s, g, be, w4, b4c)


def kernel(x, table, W1, b1, g1, be1, W2, b2, g2, be2, W3, b3, g3, be3,
           W4, b4):
    bf = jnp.bfloat16
    h = _pool(x, table)
    z1, s1 = _k1(h, W1.T, b1.reshape(1, H))
    z2, s2 = _klayer(z1, s1, g1.reshape(1, H), be1.reshape(1, H),
                     W2.T.astype(bf), b2.reshape(1, H))
    z3, s3 = _klayer(z2, s2, g2.reshape(1, H), be2.reshape(1, H),
                     W3.T.astype(bf), b3.reshape(1, H))
    ot = _kfinal(z3, s3, g3.reshape(1, H), be3.reshape(1, H),
                 W4.astype(bf), b4.reshape(C, 1))
    return ot.T


# speedup vs baseline: 13.0015x; 1.0006x over previous
import functools

import jax
import jax.numpy as jnp
from jax import lax
from jax.experimental import pallas as pl
from jax.experimental.pallas import tpu as pltpu
from jax.experimental.pallas import tpu_sc as plsc

MAXW = 1000000
D = 32
H = 1000
C = 1000
B = 16384
L = 50
EPS = 1e-5

NC = 2
NS = 16
NW = NC * NS
BPW = B // NW
NT = BPW
NBUF = 8


def _pool_body(x_hbm, table_hbm, h_hbm, idx_v, rows_v, h_v, sems):
    wid = lax.axis_index("s") * NC + lax.axis_index("c")
    pltpu.sync_copy(x_hbm.at[pl.ds(wid * BPW, BPW)], idx_v)

    def fire(t, b):
        pltpu.async_copy(table_hbm.at[idx_v.at[t]], rows_v.at[b], sems.at[b])

    for b in range(NBUF):
        fire(b, b)

    def outer(g, carry):
        for b in range(NBUF):
            t = g * NBUF + b
            pltpu.make_async_copy(
                table_hbm.at[idx_v.at[0]], rows_v.at[b], sems.at[b]).wait()
            acc0 = rows_v[b, 0, pl.ds(0, 16)]
            acc1 = rows_v[b, 0, pl.ds(16, 16)]
            for l in range(1, L):
                acc0 = acc0 + rows_v[b, l, pl.ds(0, 16)]
                acc1 = acc1 + rows_v[b, l, pl.ds(16, 16)]
            h_v[t, pl.ds(0, 16)] = acc0
            h_v[t, pl.ds(16, 16)] = acc1
            tn = t + NBUF

            @pl.when(tn < NT)
            def _():
                fire(tn, b)
        return carry

    lax.fori_loop(0, NT // NBUF, outer, 0)
    pltpu.sync_copy(h_v, h_hbm.at[pl.ds(wid * BPW, BPW)])


@functools.partial(jax.jit, static_argnames=())
def _pool(x, table):
    mesh = plsc.VectorSubcoreMesh(core_axis_name="c", subcore_axis_name="s")
    return pl.kernel(
        _pool_body,
        out_type=jax.ShapeDtypeStruct((B, D), jnp.float32),
        mesh=mesh,
        scratch_types=[
            pltpu.VMEM((BPW, L), jnp.int32),
            pltpu.VMEM((NBUF, L, D), jnp.float32),
            pltpu.VMEM((BPW, D), jnp.float32),
            pltpu.SemaphoreType.DMA((NBUF,)),
        ],
        compiler_params=pltpu.CompilerParams(use_tc_tiling_on_sc=False),
    )(x, table)


BB1 = 2048
BBL = 2048


def _k1_body(h_ref, w_ref, b_ref, z_ref, s_ref):
    z = jnp.dot(h_ref[...], w_ref[...],
                preferred_element_type=jnp.float32) + b_ref[...]
    z_ref[...] = z.astype(jnp.bfloat16)
    acc = jnp.concatenate(
        [jnp.sum(z, axis=0, keepdims=True),
         jnp.sum(z * z, axis=0, keepdims=True)], axis=0)

    @pl.when(pl.program_id(0) == 0)
    def _():
        s_ref[...] = acc

    @pl.when(pl.program_id(0) != 0)
    def _():
        s_ref[...] += acc


def _layer_body(z_ref, s_ref, g_ref, be_ref, w_ref, b_ref, zn_ref, so_ref):
    s = s_ref[...]
    mu = s[0:1, :] * (1.0 / B)
    var = s[1:2, :] * (1.0 / B) - mu * mu
    inv = g_ref[...] * lax.rsqrt(var + EPS)
    a = jnp.maximum((z_ref[...].astype(jnp.float32) - mu) * inv
                    + be_ref[...], 0.0)
    zn = jnp.dot(a.astype(jnp.bfloat16), w_ref[...],
                 preferred_element_type=jnp.float32) + b_ref[...]
    zn_ref[...] = zn.astype(jnp.bfloat16)
    acc = jnp.concatenate(
        [jnp.sum(zn, axis=0, keepdims=True),
         jnp.sum(zn * zn, axis=0, keepdims=True)], axis=0)

    @pl.when(pl.program_id(0) == 0)
    def _():
        so_ref[...] = acc

    @pl.when(pl.program_id(0) != 0)
    def _():
        so_ref[...] += acc


def _final_body(z_ref, s_ref, g_ref, be_ref, w_ref, b_ref, o_ref):
    s = s_ref[...]
    mu = s[0:1, :] * (1.0 / B)
    var = s[1:2, :] * (1.0 / B) - mu * mu
    inv = g_ref[...] * lax.rsqrt(var + EPS)
    a = jnp.maximum((z_ref[...].astype(jnp.float32) - mu) * inv
                    + be_ref[...], 0.0)
    ot = lax.dot_general(w_ref[...], a.astype(jnp.bfloat16),
                         (((1,), (1,)), ((), ())),
                         preferred_element_type=jnp.float32)
    o_ref[...] = ot + b_ref[...]


def _row_spec(bb, cols):
    return pl.BlockSpec((bb, cols), lambda i: (i, 0))


def _full_spec(rows, cols):
    return pl.BlockSpec((rows, cols), lambda i: (0, 0))


def _k1(h, w1t, b1):
    return pl.pallas_call(
        _k1_body,
        grid=(B // BB1,),
        in_specs=[_row_spec(BB1, D), _full_spec(D, H), _full_spec(1, H)],
        out_specs=[_row_spec(BB1, H), _full_spec(2, H)],
        out_shape=[jax.ShapeDtypeStruct((B, H), jnp.bfloat16),
                   jax.ShapeDtypeStruct((2, H), jnp.float32)],
    )(h, w1t, b1)


def _klayer(z, s, g, be, wt, b):
    return pl.pallas_call(
        _layer_body,
        grid=(B // BBL,),
        in_specs=[_row_spec(BBL, H), _full_spec(2, H), _full_spec(1, H),
                  _full_spec(1, H), _full_spec(H, H), _full_spec(1, H)],
        out_specs=[_row_spec(BBL, H), _full_spec(2, H)],
        out_shape=[jax.ShapeDtypeStruct((B, H), jnp.bfloat16),
                   jax.ShapeDtypeStruct((2, H), jnp.float32)],
    )(z, s, g, be, wt, b)


def _kfinal(z, s, g, be, w4, b4c):
    return pl.pallas_call(
        _final_body,
        grid=(B // BBL,),
        in_specs=[_row_spec(BBL, H), _full_spec(2, H), _full_spec(1, H),
                  _full_spec(1, H), _full_spec(C, H), _full_spec(C, 1)],
        out_specs=pl.BlockSpec((C, BBL), lambda i: (0, i)),
        out_shape=jax.ShapeDtypeStruct((C, B), jnp.float32),
    )(z, s, g, be, w4, b4c)


def kernel(x, table, W1, b1, g1, be1, W2, b2, g2, be2, W3, b3, g3, be3,
           W4, b4):
    bf = jnp.bfloat16
    h = _pool(x, table)
    z1, s1 = _k1(h, W1.T, b1.reshape(1, H))
    z2, s2 = _klayer(z1, s1, g1.reshape(1, H), be1.reshape(1, H),
                     W2.T.astype(bf), b2.reshape(1, H))
    z3, s3 = _klayer(z2, s2, g2.reshape(1, H), be2.reshape(1, H),
                     W3.T.astype(bf), b3.reshape(1, H))
    ot = _kfinal(z3, s3, g3.reshape(1, H), be3.reshape(1, H),
                 W4.astype(bf), b4.reshape(C, 1))
    return ot.T
